# speedup vs baseline: 3.8297x; 3.8297x over previous
import functools

import jax
import jax.numpy as jnp
from jax import lax
from jax.experimental import pallas as pl
from jax.experimental.pallas import tpu as pltpu
from jax.experimental.pallas import tpu_sc as plsc

N_NODES = 10000
N_EDGES = 320000
D = 128


def _node_proj(nfeat, W_s, W_d):
    NB = 2000

    def body(nf_ref, ws_ref, wd_ref, ps_ref, pd_ref):
        x = nf_ref[...]
        dn = (((1,), (1,)), ((), ()))
        ps_ref[...] = lax.dot_general(x, ws_ref[...], dn,
                                      preferred_element_type=jnp.float32)
        pd_ref[...] = lax.dot_general(x, wd_ref[...], dn,
                                      preferred_element_type=jnp.float32)

    return pl.pallas_call(
        body,
        grid=(N_NODES // NB,),
        in_specs=[
            pl.BlockSpec((NB, D), lambda i: (i, 0)),
            pl.BlockSpec((D, D), lambda i: (0, 0)),
            pl.BlockSpec((D, D), lambda i: (0, 0)),
        ],
        out_specs=[
            pl.BlockSpec((NB, D), lambda i: (i, 0)),
            pl.BlockSpec((NB, D), lambda i: (i, 0)),
        ],
        out_shape=[jax.ShapeDtypeStruct((N_NODES, D), jnp.float32)] * 2,
    )(nfeat, W_s, W_d)


@functools.cache
def _make_gather_add():
    info = plsc.get_sparse_core_info()
    NC, NS, L = info.num_cores, info.num_subcores, info.num_lanes
    NW = NC * NS
    per_w = N_EDGES // NW
    CH = 128
    n_full = per_w // CH
    tail = per_w - n_full * CH

    mesh = plsc.VectorSubcoreMesh(core_axis_name="c", subcore_axis_name="s")

    @functools.partial(
        pl.kernel,
        mesh=mesh,
        out_type=jax.ShapeDtypeStruct((N_EDGES, D), jnp.float32),
        scratch_types=[
            pltpu.VMEM((per_w,), jnp.int32),
            pltpu.VMEM((per_w,), jnp.int32),
            pltpu.VMEM((CH, D), jnp.float32),
            pltpu.VMEM((CH, D), jnp.float32),
            pltpu.SemaphoreType.DMA,
            pltpu.SemaphoreType.DMA,
        ],
    )
    def gather_add(ps_hbm, pd_hbm, src_hbm, dst_hbm, out_hbm,
                   idx_s, idx_d, rows_s, rows_d, sem_s, sem_d):
        wid = lax.axis_index("s") * NC + lax.axis_index("c")
        base = wid * per_w
        pltpu.sync_copy(src_hbm.at[pl.ds(base, per_w)], idx_s)
        pltpu.sync_copy(dst_hbm.at[pl.ds(base, per_w)], idx_d)

        def do_chunk(off, n):
            cs = pltpu.async_copy(ps_hbm.at[idx_s.at[pl.ds(off, n)]],
                                  rows_s.at[pl.ds(0, n)], sem_s)
            cd = pltpu.async_copy(pd_hbm.at[idx_d.at[pl.ds(off, n)]],
                                  rows_d.at[pl.ds(0, n)], sem_d)
            cs.wait()
            cd.wait()

            def row_add(r, carry):
                for j in range(D // L):
                    sl = (r, pl.ds(j * L, L))
                    rows_s[sl] = rows_s[sl] + rows_d[sl]
                return carry

            lax.fori_loop(0, n, row_add, 0)
            pltpu.sync_copy(rows_s.at[pl.ds(0, n)],
                            out_hbm.at[pl.ds(base + off, n)])

        def chunk_body(c, carry):
            do_chunk(c * CH, CH)
            return carry

        lax.fori_loop(0, n_full, chunk_body, 0)
        do_chunk(n_full * CH, tail)

    return gather_add


def _edge_mlp(efeat, g, W_e, W_f, b1, b_f, ln_g, ln_b):
    EB = 2000

    def body(e_ref, g_ref, we_ref, wf_ref, b1_ref, bf_ref, lng_ref, lnb_ref,
             o_ref):
        e = e_ref[...]
        dn = (((1,), (1,)), ((), ()))
        h = lax.dot_general(e, we_ref[...], dn,
                            preferred_element_type=jnp.float32)
        h = h + g_ref[...] + b1_ref[...]
        h = h * jax.nn.sigmoid(h)
        o = lax.dot_general(h, wf_ref[...], dn,
                            preferred_element_type=jnp.float32) + bf_ref[...]
        mu = jnp.mean(o, axis=1, keepdims=True)
        var = jnp.mean((o - mu) * (o - mu), axis=1, keepdims=True)
        o = (o - mu) * lax.rsqrt(var + 1e-5) * lng_ref[...] + lnb_ref[...]
        o_ref[...] = o + e

    vec = pl.BlockSpec((1, D), lambda i: (0, 0))
    return pl.pallas_call(
        body,
        grid=(N_EDGES // EB,),
        in_specs=[
            pl.BlockSpec((EB, D), lambda i: (i, 0)),
            pl.BlockSpec((EB, D), lambda i: (i, 0)),
            pl.BlockSpec((D, D), lambda i: (0, 0)),
            pl.BlockSpec((D, D), lambda i: (0, 0)),
            vec, vec, vec, vec,
        ],
        out_specs=pl.BlockSpec((EB, D), lambda i: (i, 0)),
        out_shape=jax.ShapeDtypeStruct((N_EDGES, D), jnp.float32),
    )(efeat, g, W_e, W_f, b1.reshape(1, D), b_f.reshape(1, D),
      ln_g.reshape(1, D), ln_b.reshape(1, D))


def kernel(efeat, nfeat, edge_index, W_e, W_s, W_d, b1, W_f, b_f, ln_g, ln_b):
    src = edge_index[0]
    dst = edge_index[1]
    ps, pd = _node_proj(nfeat, W_s, W_d)
    g = _make_gather_add()(ps, pd, src, dst)
    out = _edge_mlp(efeat, g, W_e, W_f, b1, b_f, ln_g, ln_b)
    return (out, nfeat)


# speedup vs baseline: 4.9036x; 1.2804x over previous
import functools

import jax
import jax.numpy as jnp
from jax import lax
from jax.experimental import pallas as pl
from jax.experimental.pallas import tpu as pltpu
from jax.experimental.pallas import tpu_sc as plsc

N_NODES = 10000
N_EDGES = 320000
D = 128


def _node_proj(nfeat, W_s, W_d):
    NB = 2000

    def body(nf_ref, ws_ref, wd_ref, ps_ref, pd_ref):
        x = nf_ref[...]
        dn = (((1,), (1,)), ((), ()))
        ps_ref[...] = lax.dot_general(x, ws_ref[...], dn,
                                      preferred_element_type=jnp.float32)
        pd_ref[...] = lax.dot_general(x, wd_ref[...], dn,
                                      preferred_element_type=jnp.float32)

    return pl.pallas_call(
        body,
        grid=(N_NODES // NB,),
        in_specs=[
            pl.BlockSpec((NB, D), lambda i: (i, 0)),
            pl.BlockSpec((D, D), lambda i: (0, 0)),
            pl.BlockSpec((D, D), lambda i: (0, 0)),
        ],
        out_specs=[
            pl.BlockSpec((NB, D), lambda i: (i, 0)),
            pl.BlockSpec((NB, D), lambda i: (i, 0)),
        ],
        out_shape=[jax.ShapeDtypeStruct((N_NODES, D), jnp.float32)] * 2,
    )(nfeat, W_s, W_d)


@functools.cache
def _make_gather_add():
    info = plsc.get_sparse_core_info()
    NC, NS, L = info.num_cores, info.num_subcores, info.num_lanes
    NW = NC * NS
    per_w = N_EDGES // NW
    CH = 80
    n_ch = per_w // CH
    NSL = 4

    mesh = plsc.VectorSubcoreMesh(core_axis_name="c", subcore_axis_name="s")

    @functools.partial(
        pl.kernel,
        mesh=mesh,
        out_type=jax.ShapeDtypeStruct((N_EDGES, D), jnp.float32),
        scratch_types=(
            [pltpu.VMEM((per_w,), jnp.int32)] * 2
            + [pltpu.VMEM((CH, D), jnp.float32)] * (2 * NSL)
            + [pltpu.SemaphoreType.DMA] * (2 * NSL)
        ),
    )
    def gather_add(ps_hbm, pd_hbm, src_hbm, dst_hbm, out_hbm, *refs):
        idx_s, idx_d = refs[0], refs[1]
        rows_s = refs[2:2 + NSL]
        rows_d = refs[2 + NSL:2 + 2 * NSL]
        sem_g = refs[2 + 2 * NSL:2 + 3 * NSL]
        sem_w = refs[2 + 3 * NSL:2 + 4 * NSL]

        wid = lax.axis_index("s") * NC + lax.axis_index("c")
        base = wid * per_w
        pltpu.sync_copy(src_hbm.at[pl.ds(base, per_w)], idx_s)
        pltpu.sync_copy(dst_hbm.at[pl.ds(base, per_w)], idx_d)

        def g_start(c, b):
            off = c * CH
            pltpu.make_async_copy(ps_hbm.at[idx_s.at[pl.ds(off, CH)]],
                                  rows_s[b], sem_g[b]).start()
            pltpu.make_async_copy(pd_hbm.at[idx_d.at[pl.ds(off, CH)]],
                                  rows_d[b], sem_g[b]).start()

        def g_wait(b):
            pltpu.make_async_copy(ps_hbm.at[idx_s.at[pl.ds(0, CH)]],
                                  rows_s[b], sem_g[b]).wait()
            pltpu.make_async_copy(pd_hbm.at[idx_d.at[pl.ds(0, CH)]],
                                  rows_d[b], sem_g[b]).wait()

        def wb_start(c, b):
            pltpu.make_async_copy(rows_s[b],
                                  out_hbm.at[pl.ds(base + c * CH, CH)],
                                  sem_w[b]).start()

        def wb_wait(b):
            pltpu.make_async_copy(rows_s[b],
                                  out_hbm.at[pl.ds(base, CH)],
                                  sem_w[b]).wait()

        def add(b):
            rs, rd = rows_s[b], rows_d[b]

            def row_add(r, carry):
                for j in range(D // L):
                    sl = (r, pl.ds(j * L, L))
                    rs[sl] = rs[sl] + rd[sl]
                return carry

            lax.fori_loop(0, CH, row_add, 0)

        g_start(0, 0)
        g_start(1, 1)
        g_start(2, 2)
        g_wait(0); add(0); wb_start(0, 0)
        g_start(3, 3)
        g_wait(1); add(1); wb_start(1, 1)

        def main_body(i, carry):
            c0 = 2 + i * NSL
            for b in range(NSL):
                c = c0 + b
                sg = b
                sc = (2 + b) % NSL
                wb_wait(sg)
                g_start(c + 2, sg)
                g_wait(sc); add(sc); wb_start(c, sc)
            return carry

        lax.fori_loop(0, (n_ch - 5) // NSL, main_body, 0)

        wb_wait(0)
        g_start(124, 0)
        g_wait(2); add(2); wb_start(122, 2)
        g_wait(3); add(3); wb_start(123, 3)
        g_wait(0); add(0); wb_start(124, 0)
        wb_wait(1); wb_wait(2); wb_wait(3); wb_wait(0)

    return gather_add


def _edge_mlp(efeat, g, W_e, W_f, b1, b_f, ln_g, ln_b):
    EB = 2000

    def body(e_ref, g_ref, we_ref, wf_ref, b1_ref, bf_ref, lng_ref, lnb_ref,
             o_ref):
        e = e_ref[...]
        dn = (((1,), (1,)), ((), ()))
        h = lax.dot_general(e, we_ref[...], dn,
                            preferred_element_type=jnp.float32)
        h = h + g_ref[...] + b1_ref[...]
        h = h * jax.nn.sigmoid(h)
        o = lax.dot_general(h, wf_ref[...], dn,
                            preferred_element_type=jnp.float32) + bf_ref[...]
        mu = jnp.mean(o, axis=1, keepdims=True)
        var = jnp.mean((o - mu) * (o - mu), axis=1, keepdims=True)
        o = (o - mu) * lax.rsqrt(var + 1e-5) * lng_ref[...] + lnb_ref[...]
        o_ref[...] = o + e

    vec = pl.BlockSpec((1, D), lambda i: (0, 0))
    return pl.pallas_call(
        body,
        grid=(N_EDGES // EB,),
        in_specs=[
            pl.BlockSpec((EB, D), lambda i: (i, 0)),
            pl.BlockSpec((EB, D), lambda i: (i, 0)),
            pl.BlockSpec((D, D), lambda i: (0, 0)),
            pl.BlockSpec((D, D), lambda i: (0, 0)),
            vec, vec, vec, vec,
        ],
        out_specs=pl.BlockSpec((EB, D), lambda i: (i, 0)),
        out_shape=jax.ShapeDtypeStruct((N_EDGES, D), jnp.float32),
    )(efeat, g, W_e, W_f, b1.reshape(1, D), b_f.reshape(1, D),
      ln_g.reshape(1, D), ln_b.reshape(1, D))


def kernel(efeat, nfeat, edge_index, W_e, W_s, W_d, b1, W_f, b_f, ln_g, ln_b):
    src = edge_index[0]
    dst = edge_index[1]
    ps, pd = _node_proj(nfeat, W_s, W_d)
    g = _make_gather_add()(ps, pd, src, dst)
    out = _edge_mlp(efeat, g, W_e, W_f, b1, b_f, ln_g, ln_b)
    return (out, nfeat)


# speedup vs baseline: 5.6052x; 1.1431x over previous
import functools

import jax
import jax.numpy as jnp
from jax import lax
from jax.experimental import pallas as pl
from jax.experimental.pallas import tpu as pltpu
from jax.experimental.pallas import tpu_sc as plsc

N_NODES = 10000
N_EDGES = 320000
D = 128


def _node_proj(nfeat, W_s, W_d):
    NB = 2000

    def body(nf_ref, ws_ref, wd_ref, ps_ref, pd_ref):
        x = nf_ref[...]
        dn = (((1,), (1,)), ((), ()))
        ps_ref[...] = lax.dot_general(x, ws_ref[...], dn,
                                      preferred_element_type=jnp.float32)
        pd_ref[...] = lax.dot_general(x, wd_ref[...], dn,
                                      preferred_element_type=jnp.float32)

    return pl.pallas_call(
        body,
        grid=(N_NODES // NB,),
        in_specs=[
            pl.BlockSpec((NB, D), lambda i: (i, 0)),
            pl.BlockSpec((D, D), lambda i: (0, 0)),
            pl.BlockSpec((D, D), lambda i: (0, 0)),
        ],
        out_specs=[
            pl.BlockSpec((NB, D), lambda i: (i, 0)),
            pl.BlockSpec((NB, D), lambda i: (i, 0)),
        ],
        out_shape=[jax.ShapeDtypeStruct((N_NODES, D), jnp.float32)] * 2,
    )(nfeat, W_s, W_d)


@functools.cache
def _make_gather_add():
    info = plsc.get_sparse_core_info()
    NC, NS, L = info.num_cores, info.num_subcores, info.num_lanes
    NW = NC * NS
    per_w = N_EDGES // NW
    CH = 80
    n_ch = per_w // CH
    NSL = 4

    mesh = plsc.VectorSubcoreMesh(core_axis_name="c", subcore_axis_name="s")

    @functools.partial(
        pl.kernel,
        mesh=mesh,
        out_type=jax.ShapeDtypeStruct((N_EDGES, D), jnp.float32),
        scratch_types=(
            [pltpu.VMEM((per_w,), jnp.int32)] * 2
            + [pltpu.VMEM((CH, D), jnp.float32)] * (2 * NSL)
            + [pltpu.SemaphoreType.DMA] * (2 * NSL)
        ),
    )
    def gather_add(ps_hbm, pd_hbm, src_hbm, dst_hbm, out_hbm, *refs):
        idx_s, idx_d = refs[0], refs[1]
        rows_s = refs[2:2 + NSL]
        rows_d = refs[2 + NSL:2 + 2 * NSL]
        sem_g = refs[2 + 2 * NSL:2 + 3 * NSL]
        sem_w = refs[2 + 3 * NSL:2 + 4 * NSL]

        wid = lax.axis_index("s") * NC + lax.axis_index("c")
        base = wid * per_w
        pltpu.sync_copy(src_hbm.at[pl.ds(base, per_w)], idx_s)
        pltpu.sync_copy(dst_hbm.at[pl.ds(base, per_w)], idx_d)

        def g_start(c, b):
            off = c * CH
            pltpu.make_async_copy(ps_hbm.at[idx_s.at[pl.ds(off, CH)]],
                                  rows_s[b], sem_g[b]).start()
            pltpu.make_async_copy(pd_hbm.at[idx_d.at[pl.ds(off, CH)]],
                                  rows_d[b], sem_g[b]).start()

        def g_wait(b):
            pltpu.make_async_copy(ps_hbm.at[idx_s.at[pl.ds(0, CH)]],
                                  rows_s[b], sem_g[b]).wait()
            pltpu.make_async_copy(pd_hbm.at[idx_d.at[pl.ds(0, CH)]],
                                  rows_d[b], sem_g[b]).wait()

        def wb_start(c, b):
            pltpu.make_async_copy(rows_s[b],
                                  out_hbm.at[pl.ds(base + c * CH, CH)],
                                  sem_w[b]).start()

        def wb_wait(b):
            pltpu.make_async_copy(rows_s[b],
                                  out_hbm.at[pl.ds(base, CH)],
                                  sem_w[b]).wait()

        def add(b):
            rs, rd = rows_s[b], rows_d[b]

            def row_add(r, carry):
                for j in range(D // L):
                    sl = (r, pl.ds(j * L, L))
                    rs[sl] = rs[sl] + rd[sl]
                return carry

            lax.fori_loop(0, CH, row_add, 0)

        g_start(0, 0)
        g_start(1, 1)
        g_start(2, 2)
        g_wait(0); add(0); wb_start(0, 0)
        g_start(3, 3)
        g_wait(1); add(1); wb_start(1, 1)

        def main_body(i, carry):
            c0 = 2 + i * NSL
            for b in range(NSL):
                c = c0 + b
                sg = b
                sc = (2 + b) % NSL
                wb_wait(sg)
                g_start(c + 2, sg)
                g_wait(sc); add(sc); wb_start(c, sc)
            return carry

        lax.fori_loop(0, (n_ch - 5) // NSL, main_body, 0)

        wb_wait(0)
        g_start(124, 0)
        g_wait(2); add(2); wb_start(122, 2)
        g_wait(3); add(3); wb_start(123, 3)
        g_wait(0); add(0); wb_start(124, 0)
        wb_wait(1); wb_wait(2); wb_wait(3); wb_wait(0)

    return gather_add


def _edge_mlp(efeat, g, W_e, W_f, b1, b_f, ln_g, ln_b):
    EB = 4000

    def body(e_ref, g_ref, we_ref, wf_ref, b1_ref, bf_ref, lng_ref, lnb_ref,
             o_ref):
        e = e_ref[...]
        dn = (((1,), (1,)), ((), ()))
        h = lax.dot_general(e, we_ref[...], dn,
                            preferred_element_type=jnp.float32)
        h = h + g_ref[...] + b1_ref[...]
        h = h * jax.nn.sigmoid(h)
        o = lax.dot_general(h, wf_ref[...], dn,
                            preferred_element_type=jnp.float32) + bf_ref[...]
        mu = jnp.mean(o, axis=1, keepdims=True)
        var = jnp.mean((o - mu) * (o - mu), axis=1, keepdims=True)
        o = (o - mu) * lax.rsqrt(var + 1e-5) * lng_ref[...] + lnb_ref[...]
        o_ref[...] = o + e

    vec = pl.BlockSpec((1, D), lambda i: (0, 0))
    return pl.pallas_call(
        body,
        grid=(N_EDGES // EB,),
        in_specs=[
            pl.BlockSpec((EB, D), lambda i: (i, 0)),
            pl.BlockSpec((EB, D), lambda i: (i, 0)),
            pl.BlockSpec((D, D), lambda i: (0, 0)),
            pl.BlockSpec((D, D), lambda i: (0, 0)),
            vec, vec, vec, vec,
        ],
        out_specs=pl.BlockSpec((EB, D), lambda i: (i, 0)),
        out_shape=jax.ShapeDtypeStruct((N_EDGES, D), jnp.float32),
    )(efeat, g, W_e, W_f, b1.reshape(1, D), b_f.reshape(1, D),
      ln_g.reshape(1, D), ln_b.reshape(1, D))


def kernel(efeat, nfeat, edge_index, W_e, W_s, W_d, b1, W_f, b_f, ln_g, ln_b):
    src = edge_index[0]
    dst = edge_index[1]
    ps, pd = _node_proj(nfeat, W_s, W_d)
    g = _make_gather_add()(ps, pd, src, dst)
    out = _edge_mlp(efeat, g, W_e, W_f, b1, b_f, ln_g, ln_b)
    return (out, nfeat)


# speedup vs baseline: 5.9510x; 1.0617x over previous
import functools

import jax
import jax.numpy as jnp
from jax import lax
from jax.experimental import pallas as pl
from jax.experimental.pallas import tpu as pltpu
from jax.experimental.pallas import tpu_sc as plsc

N_NODES = 10000
N_EDGES = 320000
D = 128


def _node_proj(nfeat, W_s, W_d):
    NB = 2000

    def body(nf_ref, ws_ref, wd_ref, ps_ref, pd_ref):
        x = nf_ref[...]
        dn = (((1,), (1,)), ((), ()))
        ps_ref[...] = lax.dot_general(x, ws_ref[...], dn,
                                      preferred_element_type=jnp.float32)
        pd_ref[...] = lax.dot_general(x, wd_ref[...], dn,
                                      preferred_element_type=jnp.float32)

    return pl.pallas_call(
        body,
        grid=(N_NODES // NB,),
        in_specs=[
            pl.BlockSpec((NB, D), lambda i: (i, 0)),
            pl.BlockSpec((D, D), lambda i: (0, 0)),
            pl.BlockSpec((D, D), lambda i: (0, 0)),
        ],
        out_specs=[
            pl.BlockSpec((NB, D), lambda i: (i, 0)),
            pl.BlockSpec((NB, D), lambda i: (i, 0)),
        ],
        out_shape=[jax.ShapeDtypeStruct((N_NODES, D), jnp.float32)] * 2,
    )(nfeat, W_s, W_d)


@functools.cache
def _make_gather_add():
    info = plsc.get_sparse_core_info()
    NC, NS, L = info.num_cores, info.num_subcores, info.num_lanes
    NW = NC * NS
    per_w = N_EDGES // NW
    CH = 80
    n_ch = per_w // CH
    NSL = 4

    mesh = plsc.VectorSubcoreMesh(core_axis_name="c", subcore_axis_name="s")

    @functools.partial(
        pl.kernel,
        mesh=mesh,
        out_type=jax.ShapeDtypeStruct((N_EDGES, D), jnp.float32),
        scratch_types=(
            [pltpu.VMEM((per_w,), jnp.int32)] * 2
            + [pltpu.VMEM((CH, D), jnp.float32)] * (2 * NSL)
            + [pltpu.SemaphoreType.DMA] * (2 * NSL)
        ),
    )
    def gather_add(ps_hbm, pd_hbm, src_hbm, dst_hbm, out_hbm, *refs):
        idx_s, idx_d = refs[0], refs[1]
        rows_s = refs[2:2 + NSL]
        rows_d = refs[2 + NSL:2 + 2 * NSL]
        sem_g = refs[2 + 2 * NSL:2 + 3 * NSL]
        sem_w = refs[2 + 3 * NSL:2 + 4 * NSL]

        wid = lax.axis_index("s") * NC + lax.axis_index("c")
        base = wid * per_w
        pltpu.sync_copy(src_hbm.at[pl.ds(base, per_w)], idx_s)
        pltpu.sync_copy(dst_hbm.at[pl.ds(base, per_w)], idx_d)

        def g_start(c, b):
            off = c * CH
            pltpu.make_async_copy(ps_hbm.at[idx_s.at[pl.ds(off, CH)]],
                                  rows_s[b], sem_g[b]).start()
            pltpu.make_async_copy(pd_hbm.at[idx_d.at[pl.ds(off, CH)]],
                                  rows_d[b], sem_g[b]).start()

        def g_wait(b):
            pltpu.make_async_copy(ps_hbm.at[idx_s.at[pl.ds(0, CH)]],
                                  rows_s[b], sem_g[b]).wait()
            pltpu.make_async_copy(pd_hbm.at[idx_d.at[pl.ds(0, CH)]],
                                  rows_d[b], sem_g[b]).wait()

        def wb_start(c, b):
            pltpu.make_async_copy(rows_s[b],
                                  out_hbm.at[pl.ds(base + c * CH, CH)],
                                  sem_w[b]).start()

        def wb_wait(b):
            pltpu.make_async_copy(rows_s[b],
                                  out_hbm.at[pl.ds(base, CH)],
                                  sem_w[b]).wait()

        def add(b):
            rs, rd = rows_s[b], rows_d[b]

            def row_add(r, carry):
                for j in range(D // L):
                    sl = (r, pl.ds(j * L, L))
                    rs[sl] = rs[sl] + rd[sl]
                return carry

            lax.fori_loop(0, CH, row_add, 0)

        g_start(0, 0)
        g_start(1, 1)
        g_start(2, 2)
        g_wait(0); add(0); wb_start(0, 0)
        g_start(3, 3)
        g_wait(1); add(1); wb_start(1, 1)

        def main_body(i, carry):
            c0 = 2 + i * NSL
            for b in range(NSL):
                c = c0 + b
                sg = b
                sc = (2 + b) % NSL
                wb_wait(sg)
                g_start(c + 2, sg)
                g_wait(sc); add(sc); wb_start(c, sc)
            return carry

        lax.fori_loop(0, (n_ch - 5) // NSL, main_body, 0)

        wb_wait(0)
        g_start(124, 0)
        g_wait(2); add(2); wb_start(122, 2)
        g_wait(3); add(3); wb_start(123, 3)
        g_wait(0); add(0); wb_start(124, 0)
        wb_wait(1); wb_wait(2); wb_wait(3); wb_wait(0)

    return gather_add


def _edge_mlp(efeat, g, W_e, W_f, b1, b_f, ln_g, ln_b):
    EB = 8000

    def body(e_ref, g_ref, we_ref, wf_ref, b1_ref, bf_ref, lng_ref, lnb_ref,
             o_ref):
        e = e_ref[...]
        dn = (((1,), (1,)), ((), ()))
        h = lax.dot_general(e, we_ref[...], dn,
                            preferred_element_type=jnp.float32)
        h = h + g_ref[...] + b1_ref[...]
        h = h * jax.nn.sigmoid(h)
        o = lax.dot_general(h, wf_ref[...], dn,
                            preferred_element_type=jnp.float32) + bf_ref[...]
        mu = jnp.mean(o, axis=1, keepdims=True)
        var = jnp.mean((o - mu) * (o - mu), axis=1, keepdims=True)
        o = (o - mu) * lax.rsqrt(var + 1e-5) * lng_ref[...] + lnb_ref[...]
        o_ref[...] = o + e

    vec = pl.BlockSpec((1, D), lambda i: (0, 0))
    return pl.pallas_call(
        body,
        grid=(N_EDGES // EB,),
        in_specs=[
            pl.BlockSpec((EB, D), lambda i: (i, 0)),
            pl.BlockSpec((EB, D), lambda i: (i, 0)),
            pl.BlockSpec((D, D), lambda i: (0, 0)),
            pl.BlockSpec((D, D), lambda i: (0, 0)),
            vec, vec, vec, vec,
        ],
        out_specs=pl.BlockSpec((EB, D), lambda i: (i, 0)),
        out_shape=jax.ShapeDtypeStruct((N_EDGES, D), jnp.float32),
    )(efeat, g, W_e, W_f, b1.reshape(1, D), b_f.reshape(1, D),
      ln_g.reshape(1, D), ln_b.reshape(1, D))


def kernel(efeat, nfeat, edge_index, W_e, W_s, W_d, b1, W_f, b_f, ln_g, ln_b):
    src = edge_index[0]
    dst = edge_index[1]
    ps, pd = _node_proj(nfeat, W_s, W_d)
    g = _make_gather_add()(ps, pd, src, dst)
    out = _edge_mlp(efeat, g, W_e, W_f, b1, b_f, ln_g, ln_b)
    return (out, nfeat)
